# Initial kernel scaffold; baseline (speedup 1.0000x reference)
import jax
import jax.numpy as jnp
from jax.experimental import pallas as pl


def kernel(loc_data, conf_data, targets, anchor_boxes):
    raise NotImplementedError("write your pallas kernel here")


# speedup vs baseline: 13.8999x; 13.8999x over previous
import functools

import jax
import jax.numpy as jnp
from jax.experimental import pallas as pl

_A = 5000
_APAD = 5120
_ROWS = 40
_COLS = 128
_NOBJ = 16
_VAR0 = 0.1
_VAR1 = 0.2
_THRESH = 0.5


def _match_body(t_ref, a_ref, loc_ref, conf_ref):
    acx = a_ref[0]
    acy = a_ref[1]
    aw = a_ref[2]
    ah = a_ref[3]
    ax1 = acx - aw / 2.0
    ay1 = acy - ah / 2.0
    ax2 = acx + aw / 2.0
    ay2 = acy + ah / 2.0
    area_b = (ax2 - ax1) * (ay2 - ay1)

    row = jax.lax.broadcasted_iota(jnp.int32, (_ROWS, _COLS), 0)
    col = jax.lax.broadcasted_iota(jnp.int32, (_ROWS, _COLS), 1)
    flat = row * _COLS + col
    valid = flat < _A

    t = t_ref[0]

    best_v = jnp.full((_ROWS, _COLS), -2.0, jnp.float32)
    best_g = jnp.zeros((_ROWS, _COLS), jnp.int32)
    anchor_idx = []
    for g in range(_NOBJ):
        tx1 = t[g, 0]
        ty1 = t[g, 1]
        tx2 = t[g, 2]
        ty2 = t[g, 3]
        ltx = jnp.maximum(tx1, ax1)
        lty = jnp.maximum(ty1, ay1)
        rbx = jnp.minimum(tx2, ax2)
        rby = jnp.minimum(ty2, ay2)
        w = jnp.clip(rbx - ltx, 0.0)
        h = jnp.clip(rby - lty, 0.0)
        inter = w * h
        area_a = (tx2 - tx1) * (ty2 - ty1)
        iou = inter / (area_a + area_b - inter)
        iou = jnp.where(valid, iou, -1.0)
        if g == 0:
            best_v = iou
        else:
            upd = iou > best_v
            best_v = jnp.where(upd, iou, best_v)
            best_g = jnp.where(upd, g, best_g)
        m = jnp.max(iou)
        anchor_idx.append(jnp.min(jnp.where(iou == m, flat, _APAD)))

    for g in range(_NOBJ):
        hit = flat == anchor_idx[g]
        best_v = jnp.where(hit, 1.0, best_v)
        best_g = jnp.where(hit, g, best_g)

    gx1 = jnp.zeros((_ROWS, _COLS), jnp.float32)
    gy1 = jnp.zeros((_ROWS, _COLS), jnp.float32)
    gx2 = jnp.zeros((_ROWS, _COLS), jnp.float32)
    gy2 = jnp.zeros((_ROWS, _COLS), jnp.float32)
    glb = jnp.zeros((_ROWS, _COLS), jnp.float32)
    for g in range(_NOBJ):
        sel = best_g == g
        gx1 = jnp.where(sel, t[g, 0], gx1)
        gy1 = jnp.where(sel, t[g, 1], gy1)
        gx2 = jnp.where(sel, t[g, 2], gx2)
        gy2 = jnp.where(sel, t[g, 3], gy2)
        glb = jnp.where(sel, t[g, 4], glb)

    gcx = (gx1 + gx2) / 2.0
    gcy = (gy1 + gy2) / 2.0
    gw = gx2 - gx1
    gh = gy2 - gy1

    loc_ref[0, 0] = (gcx - acx) / (_VAR0 * aw)
    loc_ref[0, 1] = (gcy - acy) / (_VAR0 * ah)
    loc_ref[0, 2] = jnp.log(gw / aw) / _VAR1
    loc_ref[0, 3] = jnp.log(gh / ah) / _VAR1

    conf = glb.astype(jnp.int32) + 1
    conf_ref[0] = jnp.where(best_v < _THRESH, 0, conf)


@functools.partial(jax.jit, static_argnames=())
def kernel(loc_data, conf_data, targets, anchor_boxes):
    del loc_data, conf_data
    b = targets.shape[0]
    a = jnp.pad(anchor_boxes.T, ((0, 0), (0, _APAD - _A))).reshape(4, _ROWS, _COLS)
    loc_out, conf_out = pl.pallas_call(
        _match_body,
        grid=(b,),
        in_specs=[
            pl.BlockSpec((1, _NOBJ, 5), lambda i: (i, 0, 0)),
            pl.BlockSpec((4, _ROWS, _COLS), lambda i: (0, 0, 0)),
        ],
        out_specs=[
            pl.BlockSpec((1, 8, _ROWS, _COLS), lambda i: (i, 0, 0, 0)),
            pl.BlockSpec((1, _ROWS, _COLS), lambda i: (i, 0, 0)),
        ],
        out_shape=[
            jax.ShapeDtypeStruct((b, 8, _ROWS, _COLS), jnp.float32),
            jax.ShapeDtypeStruct((b, _ROWS, _COLS), jnp.int32),
        ],
    )(targets, a)
    loc = loc_out[:, :4].reshape(b, 4, _APAD)[:, :, :_A].transpose(0, 2, 1)
    conf = conf_out.reshape(b, _APAD)[:, :_A]
    return loc, conf
